# baseline (speedup 1.0000x reference)
import functools

import jax
import jax.numpy as jnp
from jax import lax
from jax.experimental import pallas as pl
from jax.experimental.pallas import tpu as pltpu
from jax.experimental.pallas import tpu_sc as plsc

N = 10000
E = 320000
D = 128
NC = 2
NS = 16
NW = NC * NS
EPW = E // NW
CH = 128
NCHUNK = 80
EP = NW * NCHUNK * CH
SLOW_C = 1
NCH_S = 54
NCH_F = 104
EP2 = (NCH_S + NCH_F) * NS * CH
NP = 10240
RPT = NP // NS


def _mesh():
  return plsc.VectorSubcoreMesh(core_axis_name="c", subcore_axis_name="s",
                                num_cores=NC, num_subcores=NS)


@functools.lru_cache(maxsize=None)
def _make_seg_kernel():
  scratch = (
      pltpu.VMEM((CH,), jnp.int32),
      pltpu.VMEM((CH,), jnp.int32),
      pltpu.VMEM((CH,), jnp.int32),
      pltpu.VMEM((CH,), jnp.int32),
      pltpu.VMEM((CH, D), jnp.float32),
      pltpu.VMEM((CH, D), jnp.float32),
      pltpu.VMEM_SHARED((NP, D), jnp.float32),
      pltpu.SemaphoreType.DMA,
      pltpu.SemaphoreType.DMA,
  )

  def body(feats, src1, dst1, zrow, out, sv0, sv1, dv0, dv1, rows0, rows1,
           acc, g0, g1):
    c = lax.axis_index("c")
    s = lax.axis_index("s")
    rbase = pl.multiple_of(s * RPT, 8)
    pltpu.sync_copy(zrow.at[pl.ds(rbase, RPT)], acc.at[pl.ds(rbase, RPT)])
    plsc.subcore_barrier()

    slow = c == SLOW_C
    nch = jnp.where(slow, NCH_S, NCH_F)
    ebase = jnp.where(slow, s * (NCH_S * CH),
                      NS * (NCH_S * CH) + s * (NCH_F * CH))

    def load_idx(j, sv, dv):
      base = pl.multiple_of(ebase + j * CH, 8)
      pltpu.sync_copy(src1.at[pl.ds(base, CH)], sv)
      pltpu.sync_copy(dst1.at[pl.ds(base, CH)], dv)

    load_idx(0, sv0, dv0)
    pltpu.async_copy(feats.at[sv0], rows0, g0)

    @pl.loop(0, NCH_F, step=2)
    def step(j):
      @pl.when(j < nch)
      def _():
        load_idx(j + 1, sv1, dv1)
        pltpu.make_async_copy(feats.at[sv0], rows0, g0).wait()
        pltpu.async_copy(feats.at[sv1], rows1, g1)
        pltpu.sync_copy(rows0, acc.at[dv0], add=True)

        @pl.when(j + 2 < nch)
        def _():
          load_idx(j + 2, sv0, dv0)

        pltpu.make_async_copy(feats.at[sv1], rows1, g1).wait()

        @pl.when(j + 2 < nch)
        def _():
          pltpu.async_copy(feats.at[sv0], rows0, g0)

        pltpu.sync_copy(rows1, acc.at[dv1], add=True)

    plsc.subcore_barrier()
    pltpu.sync_copy(acc.at[pl.ds(rbase, RPT)], out.at[c, pl.ds(rbase, RPT)])

  return pl.kernel(
      body, out_type=jax.ShapeDtypeStruct((NC, NP, D), jnp.float32),
      mesh=_mesh(), scratch_types=scratch)


@functools.lru_cache(maxsize=None)
def _make_cnt_kernel():
  scratch = (
      pltpu.VMEM((NCHUNK, CH), jnp.int32),
      pltpu.VMEM((CH, D), jnp.float32),
      pltpu.VMEM_SHARED((NP, D), jnp.float32),
  )

  def body(dst3, zrow, out, didx, ones_v, cacc):
    c = lax.axis_index("c")
    s = lax.axis_index("s")
    wid = s * NC + c
    rbase = pl.multiple_of(s * RPT, 8)

    @pl.loop(0, CH)
    def fill_row(i):
      @pl.loop(0, D // 16)
      def fill_lane(k):
        ones_v[i, pl.ds(k * 16, 16)] = jnp.ones((16,), jnp.float32)

    pltpu.sync_copy(zrow.at[pl.ds(rbase, RPT)], cacc.at[pl.ds(rbase, RPT)])
    pltpu.sync_copy(dst3.at[wid], didx)
    plsc.subcore_barrier()

    @pl.loop(0, NCHUNK)
    def chunk(j):
      pltpu.sync_copy(ones_v, cacc.at[didx.at[j]], add=True)

    plsc.subcore_barrier()
    pltpu.sync_copy(cacc.at[pl.ds(rbase, RPT)], out.at[c, pl.ds(rbase, RPT)])

  return pl.kernel(
      body, out_type=jax.ShapeDtypeStruct((NC, NP, D), jnp.float32),
      mesh=_mesh(), scratch_types=scratch)


def _dotT(a, w):
  return lax.dot_general(a, w, (((1,), (1,)), ((), ())),
                         preferred_element_type=jnp.float32)


def _rownorm(o):
  nrm = jnp.maximum(jnp.sqrt(jnp.sum(o * o, axis=1, keepdims=True)), 1e-12)
  return o / nrm


def _leaky(h):
  return jnp.where(h >= 0, h, 0.2 * h)


def _bn(h, g, b):
  m = jnp.mean(h, axis=0, keepdims=True)
  v = jnp.mean((h - m) ** 2, axis=0, keepdims=True)
  return (h - m) / jnp.sqrt(v + 1e-5) * g + b


def _tc1_body(x, wi, bi, g1, b1, o):
  h = _leaky(_dotT(x[...], wi[...]) + bi[...])
  o[...] = _bn(h, g1[...], b1[...])


_tc1 = pl.pallas_call(
    _tc1_body,
    out_shape=jax.ShapeDtypeStruct((N, D), jnp.float32),
)


def _tc2_body(p, cnt, h, wl, bl, wr, o):
  cnt0 = jnp.maximum(cnt[0, :N, 0:1] + cnt[1, :N, 0:1], 1.0)
  agg = (p[0, :N] + p[1, :N]) / cnt0
  out = _dotT(agg, wl[...]) + bl[...] + _dotT(h[...], wr[...])
  o[...] = _leaky(_rownorm(out))


_tc2 = pl.pallas_call(
    _tc2_body,
    out_shape=jax.ShapeDtypeStruct((N, D), jnp.float32),
)


def _tc3_body(p, cnt, x1, wl, bl, wr, xs, ws, bs, g2, b2, o):
  cnt0 = jnp.maximum(cnt[0, :N, 0:1] + cnt[1, :N, 0:1], 1.0)
  agg = (p[0, :N] + p[1, :N]) / cnt0
  x2 = _rownorm(_dotT(agg, wl[...]) + bl[...] + _dotT(x1[...], wr[...]))
  out = x2 + _dotT(xs[...], ws[...]) + bs[...]
  o[...] = _rownorm(_bn(out, g2[...], b2[...]))


_tc3 = pl.pallas_call(
    _tc3_body,
    out_shape=jax.ShapeDtypeStruct((N, D), jnp.float32),
)


def kernel(x, edge_index, Wi, bi, g1, b1, Wl1, bl1, Wr1, Wl2, bl2, Wr2, Ws,
           bs, g2, b2):
  ei = edge_index.astype(jnp.int32)
  srcp = jnp.concatenate([ei[0], jnp.zeros((EP2 - E,), jnp.int32)])
  dstp = jnp.concatenate([ei[1], jnp.full((EP2 - E,), NP - 1, jnp.int32)])
  dst3 = jnp.concatenate([ei[1], jnp.full((EP - E,), NP - 1, jnp.int32)]
                         ).reshape(NW, NCHUNK, CH)
  zrow = jnp.zeros((NP, D), jnp.float32)
  row = lambda v: v.reshape(1, -1)

  h = _tc1(x, Wi, row(bi), row(g1), row(b1))
  cnt = _make_cnt_kernel()(dst3, zrow)
  p1 = _make_seg_kernel()(h, srcp, dstp, zrow)
  x1 = _tc2(p1, cnt, h, Wl1, row(bl1), Wr1)
  p2 = _make_seg_kernel()(x1, srcp, dstp, zrow)
  out = _tc3(p2, cnt, x1, Wl2, row(bl2), Wr2, h, Ws, row(bs), row(g2),
             row(b2))
  return out


# speedup vs baseline: 1.0390x; 1.0390x over previous
import functools

import jax
import jax.numpy as jnp
from jax import lax
from jax.experimental import pallas as pl
from jax.experimental.pallas import tpu as pltpu
from jax.experimental.pallas import tpu_sc as plsc

N = 10000
E = 320000
D = 128
NC = 2
NS = 16
NW = NC * NS
EPW = E // NW
CH = 128
NCHUNK = 80
EP = NW * NCHUNK * CH
SLOW_C = 1
NCH_S = 66
NCH_F = 92
EP2 = (NCH_S + NCH_F) * NS * CH
NP = 10240
RPT = NP // NS


def _mesh():
  return plsc.VectorSubcoreMesh(core_axis_name="c", subcore_axis_name="s",
                                num_cores=NC, num_subcores=NS)


@functools.lru_cache(maxsize=None)
def _make_seg_kernel():
  scratch = (
      pltpu.VMEM((CH,), jnp.int32),
      pltpu.VMEM((CH,), jnp.int32),
      pltpu.VMEM((CH,), jnp.int32),
      pltpu.VMEM((CH,), jnp.int32),
      pltpu.VMEM((CH, D), jnp.float32),
      pltpu.VMEM((CH, D), jnp.float32),
      pltpu.VMEM_SHARED((NP, D), jnp.float32),
      pltpu.SemaphoreType.DMA,
      pltpu.SemaphoreType.DMA,
  )

  def body(feats, src1, dst1, zrow, out, sv0, sv1, dv0, dv1, rows0, rows1,
           acc, g0, g1):
    c = lax.axis_index("c")
    s = lax.axis_index("s")
    rbase = pl.multiple_of(s * RPT, 8)
    pltpu.sync_copy(zrow.at[pl.ds(rbase, RPT)], acc.at[pl.ds(rbase, RPT)])
    plsc.subcore_barrier()

    slow = c == SLOW_C
    nch = jnp.where(slow, NCH_S, NCH_F)
    ebase = jnp.where(slow, s * (NCH_S * CH),
                      NS * (NCH_S * CH) + s * (NCH_F * CH))

    def load_idx(j, sv, dv):
      base = pl.multiple_of(ebase + j * CH, 8)
      pltpu.sync_copy(src1.at[pl.ds(base, CH)], sv)
      pltpu.sync_copy(dst1.at[pl.ds(base, CH)], dv)

    load_idx(0, sv0, dv0)
    pltpu.async_copy(feats.at[sv0], rows0, g0)

    @pl.loop(0, NCH_F, step=2)
    def step(j):
      @pl.when(j < nch)
      def _():
        load_idx(j + 1, sv1, dv1)
        pltpu.make_async_copy(feats.at[sv0], rows0, g0).wait()
        pltpu.async_copy(feats.at[sv1], rows1, g1)
        pltpu.sync_copy(rows0, acc.at[dv0], add=True)

        @pl.when(j + 2 < nch)
        def _():
          load_idx(j + 2, sv0, dv0)

        pltpu.make_async_copy(feats.at[sv1], rows1, g1).wait()

        @pl.when(j + 2 < nch)
        def _():
          pltpu.async_copy(feats.at[sv0], rows0, g0)

        pltpu.sync_copy(rows1, acc.at[dv1], add=True)

    plsc.subcore_barrier()
    pltpu.sync_copy(acc.at[pl.ds(rbase, RPT)], out.at[c, pl.ds(rbase, RPT)])

  return pl.kernel(
      body, out_type=jax.ShapeDtypeStruct((NC, NP, D), jnp.float32),
      mesh=_mesh(), scratch_types=scratch)


@functools.lru_cache(maxsize=None)
def _make_cnt_kernel():
  scratch = (
      pltpu.VMEM((NCHUNK, CH), jnp.int32),
      pltpu.VMEM((CH, D), jnp.float32),
      pltpu.VMEM_SHARED((NP, D), jnp.float32),
  )

  def body(dst3, zrow, out, didx, ones_v, cacc):
    c = lax.axis_index("c")
    s = lax.axis_index("s")
    wid = s * NC + c
    rbase = pl.multiple_of(s * RPT, 8)

    @pl.loop(0, CH)
    def fill_row(i):
      @pl.loop(0, D // 16)
      def fill_lane(k):
        ones_v[i, pl.ds(k * 16, 16)] = jnp.ones((16,), jnp.float32)

    pltpu.sync_copy(zrow.at[pl.ds(rbase, RPT)], cacc.at[pl.ds(rbase, RPT)])
    pltpu.sync_copy(dst3.at[wid], didx)
    plsc.subcore_barrier()

    @pl.loop(0, NCHUNK)
    def chunk(j):
      pltpu.sync_copy(ones_v, cacc.at[didx.at[j]], add=True)

    plsc.subcore_barrier()
    pltpu.sync_copy(cacc.at[pl.ds(rbase, RPT)], out.at[c, pl.ds(rbase, RPT)])

  return pl.kernel(
      body, out_type=jax.ShapeDtypeStruct((NC, NP, D), jnp.float32),
      mesh=_mesh(), scratch_types=scratch)


def _dotT(a, w):
  return lax.dot_general(a, w, (((1,), (1,)), ((), ())),
                         preferred_element_type=jnp.float32)


def _rownorm(o):
  nrm = jnp.maximum(jnp.sqrt(jnp.sum(o * o, axis=1, keepdims=True)), 1e-12)
  return o / nrm


def _leaky(h):
  return jnp.where(h >= 0, h, 0.2 * h)


def _bn(h, g, b):
  m = jnp.mean(h, axis=0, keepdims=True)
  v = jnp.mean((h - m) ** 2, axis=0, keepdims=True)
  return (h - m) / jnp.sqrt(v + 1e-5) * g + b


def _tc1_body(x, wi, bi, g1, b1, o):
  h = _leaky(_dotT(x[...], wi[...]) + bi[...])
  o[...] = _bn(h, g1[...], b1[...])


_tc1 = pl.pallas_call(
    _tc1_body,
    out_shape=jax.ShapeDtypeStruct((N, D), jnp.float32),
)


def _tc2_body(p, cnt, h, wl, bl, wr, o):
  cnt0 = jnp.maximum(cnt[0, :N, 0:1] + cnt[1, :N, 0:1], 1.0)
  agg = (p[0, :N] + p[1, :N]) / cnt0
  out = _dotT(agg, wl[...]) + bl[...] + _dotT(h[...], wr[...])
  o[...] = _leaky(_rownorm(out))


_tc2 = pl.pallas_call(
    _tc2_body,
    out_shape=jax.ShapeDtypeStruct((N, D), jnp.float32),
)


def _tc3_body(p, cnt, x1, wl, bl, wr, xs, ws, bs, g2, b2, o):
  cnt0 = jnp.maximum(cnt[0, :N, 0:1] + cnt[1, :N, 0:1], 1.0)
  agg = (p[0, :N] + p[1, :N]) / cnt0
  x2 = _rownorm(_dotT(agg, wl[...]) + bl[...] + _dotT(x1[...], wr[...]))
  out = x2 + _dotT(xs[...], ws[...]) + bs[...]
  o[...] = _rownorm(_bn(out, g2[...], b2[...]))


_tc3 = pl.pallas_call(
    _tc3_body,
    out_shape=jax.ShapeDtypeStruct((N, D), jnp.float32),
)


def kernel(x, edge_index, Wi, bi, g1, b1, Wl1, bl1, Wr1, Wl2, bl2, Wr2, Ws,
           bs, g2, b2):
  ei = edge_index.astype(jnp.int32)
  srcp = jnp.concatenate([ei[0], jnp.zeros((EP2 - E,), jnp.int32)])
  dstp = jnp.concatenate([ei[1], jnp.full((EP2 - E,), NP - 1, jnp.int32)])
  dst3 = jnp.concatenate([ei[1], jnp.full((EP - E,), NP - 1, jnp.int32)]
                         ).reshape(NW, NCHUNK, CH)
  zrow = jnp.zeros((NP, D), jnp.float32)
  row = lambda v: v.reshape(1, -1)

  h = _tc1(x, Wi, row(bi), row(g1), row(b1))
  cnt = _make_cnt_kernel()(dst3, zrow)
  p1 = _make_seg_kernel()(h, srcp, dstp, zrow)
  x1 = _tc2(p1, cnt, h, Wl1, row(bl1), Wr1)
  p2 = _make_seg_kernel()(x1, srcp, dstp, zrow)
  out = _tc3(p2, cnt, x1, Wl2, row(bl2), Wr2, h, Ws, row(bs), row(g2),
             row(b2))
  return out


# speedup vs baseline: 1.0514x; 1.0119x over previous
import functools

import jax
import jax.numpy as jnp
from jax import lax
from jax.experimental import pallas as pl
from jax.experimental.pallas import tpu as pltpu
from jax.experimental.pallas import tpu_sc as plsc

N = 10000
E = 320000
D = 128
NC = 2
NS = 16
NW = NC * NS
EPW = E // NW
CH = 128
NCHUNK = 80
EP = NW * NCHUNK * CH
SLOW_C = 1
NCH_S = 70
NCH_F = 88
EP2 = (NCH_S + NCH_F) * NS * CH
NP = 10240
RPT = NP // NS


def _mesh():
  return plsc.VectorSubcoreMesh(core_axis_name="c", subcore_axis_name="s",
                                num_cores=NC, num_subcores=NS)


@functools.lru_cache(maxsize=None)
def _make_seg_kernel():
  scratch = (
      pltpu.VMEM((CH,), jnp.int32),
      pltpu.VMEM((CH,), jnp.int32),
      pltpu.VMEM((CH,), jnp.int32),
      pltpu.VMEM((CH,), jnp.int32),
      pltpu.VMEM((CH, D), jnp.float32),
      pltpu.VMEM((CH, D), jnp.float32),
      pltpu.VMEM_SHARED((NP, D), jnp.float32),
      pltpu.SemaphoreType.DMA,
      pltpu.SemaphoreType.DMA,
  )

  def body(feats, src1, dst1, zrow, out, sv0, sv1, dv0, dv1, rows0, rows1,
           acc, g0, g1):
    c = lax.axis_index("c")
    s = lax.axis_index("s")
    rbase = pl.multiple_of(s * RPT, 8)
    pltpu.sync_copy(zrow.at[pl.ds(rbase, RPT)], acc.at[pl.ds(rbase, RPT)])
    plsc.subcore_barrier()

    slow = c == SLOW_C
    nch = jnp.where(slow, NCH_S, NCH_F)
    ebase = jnp.where(slow, s * (NCH_S * CH),
                      NS * (NCH_S * CH) + s * (NCH_F * CH))

    def load_idx(j, sv, dv):
      base = pl.multiple_of(ebase + j * CH, 8)
      pltpu.sync_copy(src1.at[pl.ds(base, CH)], sv)
      pltpu.sync_copy(dst1.at[pl.ds(base, CH)], dv)

    load_idx(0, sv0, dv0)
    pltpu.async_copy(feats.at[sv0], rows0, g0)

    @pl.loop(0, NCH_F, step=2)
    def step(j):
      @pl.when(j < nch)
      def _():
        load_idx(j + 1, sv1, dv1)
        pltpu.make_async_copy(feats.at[sv0], rows0, g0).wait()
        pltpu.async_copy(feats.at[sv1], rows1, g1)
        pltpu.sync_copy(rows0, acc.at[dv0], add=True)

        @pl.when(j + 2 < nch)
        def _():
          load_idx(j + 2, sv0, dv0)

        pltpu.make_async_copy(feats.at[sv1], rows1, g1).wait()

        @pl.when(j + 2 < nch)
        def _():
          pltpu.async_copy(feats.at[sv0], rows0, g0)

        pltpu.sync_copy(rows1, acc.at[dv1], add=True)

    plsc.subcore_barrier()
    pltpu.sync_copy(acc.at[pl.ds(rbase, RPT)], out.at[c, pl.ds(rbase, RPT)])

  return pl.kernel(
      body, out_type=jax.ShapeDtypeStruct((NC, NP, D), jnp.float32),
      mesh=_mesh(), scratch_types=scratch)


@functools.lru_cache(maxsize=None)
def _make_cnt_kernel():
  scratch = (
      pltpu.VMEM((NCHUNK, CH), jnp.int32),
      pltpu.VMEM((CH, D), jnp.float32),
      pltpu.VMEM_SHARED((NP, D), jnp.float32),
  )

  def body(dst3, zrow, out, didx, ones_v, cacc):
    c = lax.axis_index("c")
    s = lax.axis_index("s")
    wid = s * NC + c
    rbase = pl.multiple_of(s * RPT, 8)

    @pl.loop(0, CH)
    def fill_row(i):
      @pl.loop(0, D // 16)
      def fill_lane(k):
        ones_v[i, pl.ds(k * 16, 16)] = jnp.ones((16,), jnp.float32)

    pltpu.sync_copy(zrow.at[pl.ds(rbase, RPT)], cacc.at[pl.ds(rbase, RPT)])
    pltpu.sync_copy(dst3.at[wid], didx)
    plsc.subcore_barrier()

    @pl.loop(0, NCHUNK)
    def chunk(j):
      pltpu.sync_copy(ones_v, cacc.at[didx.at[j]], add=True)

    plsc.subcore_barrier()
    pltpu.sync_copy(cacc.at[pl.ds(rbase, RPT)], out.at[c, pl.ds(rbase, RPT)])

  return pl.kernel(
      body, out_type=jax.ShapeDtypeStruct((NC, NP, D), jnp.float32),
      mesh=_mesh(), scratch_types=scratch)


def _dotT(a, w):
  return lax.dot_general(a, w, (((1,), (1,)), ((), ())),
                         preferred_element_type=jnp.float32)


def _rownorm(o):
  nrm = jnp.maximum(jnp.sqrt(jnp.sum(o * o, axis=1, keepdims=True)), 1e-12)
  return o / nrm


def _leaky(h):
  return jnp.where(h >= 0, h, 0.2 * h)


def _bn(h, g, b):
  m = jnp.mean(h, axis=0, keepdims=True)
  v = jnp.mean((h - m) ** 2, axis=0, keepdims=True)
  return (h - m) / jnp.sqrt(v + 1e-5) * g + b


def _tc1_body(x, wi, bi, g1, b1, o):
  h = _leaky(_dotT(x[...], wi[...]) + bi[...])
  o[...] = _bn(h, g1[...], b1[...])


_tc1 = pl.pallas_call(
    _tc1_body,
    out_shape=jax.ShapeDtypeStruct((N, D), jnp.float32),
)


def _tc2_body(p, cnt, h, wl, bl, wr, o):
  cnt0 = jnp.maximum(cnt[0, :N, 0:1] + cnt[1, :N, 0:1], 1.0)
  agg = (p[0, :N] + p[1, :N]) / cnt0
  out = _dotT(agg, wl[...]) + bl[...] + _dotT(h[...], wr[...])
  o[...] = _leaky(_rownorm(out))


_tc2 = pl.pallas_call(
    _tc2_body,
    out_shape=jax.ShapeDtypeStruct((N, D), jnp.float32),
)


def _tc3_body(p, cnt, x1, wl, bl, wr, xs, ws, bs, g2, b2, o):
  cnt0 = jnp.maximum(cnt[0, :N, 0:1] + cnt[1, :N, 0:1], 1.0)
  agg = (p[0, :N] + p[1, :N]) / cnt0
  x2 = _rownorm(_dotT(agg, wl[...]) + bl[...] + _dotT(x1[...], wr[...]))
  out = x2 + _dotT(xs[...], ws[...]) + bs[...]
  o[...] = _rownorm(_bn(out, g2[...], b2[...]))


_tc3 = pl.pallas_call(
    _tc3_body,
    out_shape=jax.ShapeDtypeStruct((N, D), jnp.float32),
)


def kernel(x, edge_index, Wi, bi, g1, b1, Wl1, bl1, Wr1, Wl2, bl2, Wr2, Ws,
           bs, g2, b2):
  ei = edge_index.astype(jnp.int32)
  srcp = jnp.concatenate([ei[0], jnp.zeros((EP2 - E,), jnp.int32)])
  dstp = jnp.concatenate([ei[1], jnp.full((EP2 - E,), NP - 1, jnp.int32)])
  dst3 = jnp.concatenate([ei[1], jnp.full((EP - E,), NP - 1, jnp.int32)]
                         ).reshape(NW, NCHUNK, CH)
  zrow = jnp.zeros((NP, D), jnp.float32)
  row = lambda v: v.reshape(1, -1)

  h = _tc1(x, Wi, row(bi), row(g1), row(b1))
  cnt = _make_cnt_kernel()(dst3, zrow)
  p1 = _make_seg_kernel()(h, srcp, dstp, zrow)
  x1 = _tc2(p1, cnt, h, Wl1, row(bl1), Wr1)
  p2 = _make_seg_kernel()(x1, srcp, dstp, zrow)
  out = _tc3(p2, cnt, x1, Wl2, row(bl2), Wr2, h, Ws, row(bs), row(g2),
             row(b2))
  return out
